# Initial kernel scaffold; baseline (speedup 1.0000x reference)
import jax
import jax.numpy as jnp
from jax.experimental import pallas as pl


def kernel(x, edge_index, mp_w1, mp_b1, mp_w2, mp_b2, tag_ws, tag_bs):
    raise NotImplementedError("write your pallas kernel here")


# speedup vs baseline: 6.5744x; 6.5744x over previous
import functools

import jax
import jax.numpy as jnp
from jax import lax
from jax.experimental import pallas as pl
from jax.experimental.pallas import tpu as pltpu
from jax.experimental.pallas import tpu_sc as plsc

NC = 2
NS = 16
NW = NC * NS
L = 16
CHUNK = 128
ZR = 64


def _cdiv(a, b):
    return -(-a // b)


@functools.lru_cache(maxsize=None)
def _make_mp_sc(NP, H, CPT):
    RPT = NP // NS
    mesh = plsc.VectorSubcoreMesh(core_axis_name="c", subcore_axis_name="s")

    @functools.partial(
        pl.kernel,
        out_type=jax.ShapeDtypeStruct((NC * NP, H), jnp.float32),
        mesh=mesh,
        compiler_params=pltpu.CompilerParams(use_tc_tiling_on_sc=False),
        scratch_types=[
            pltpu.VMEM((CPT, CHUNK), jnp.int32),
            pltpu.VMEM((CPT, CHUNK), jnp.int32),
            pltpu.VMEM((CPT, CHUNK), jnp.int32),
            pltpu.VMEM((CHUNK, H), jnp.float32),
            pltpu.VMEM((CHUNK, H), jnp.float32),
            pltpu.VMEM((CHUNK, H), jnp.float32),
            pltpu.VMEM((ZR, H), jnp.float32),
            pltpu.VMEM_SHARED((NP, H), jnp.float32),
            pltpu.SemaphoreType.DMA,
            pltpu.SemaphoreType.DMA,
        ],
    )
    def mp_kernel(a_hbm, b_hbm, srcg_hbm, dstg_hbm, dstl_hbm, out_hbm,
                  srcg_v, dstg_v, dstl_v, a_v, b_v, g_v, zbuf,
                  acc, sem_a, sem_b):
        c = lax.axis_index("c")
        s = lax.axis_index("s")
        wid = c * NS + s
        zero = jnp.zeros((L,), jnp.float32)

        def zrow(i, _):
            for q in range(H // L):
                zbuf[i, pl.ds(q * L, L)] = zero
            return 0
        lax.fori_loop(0, ZR, zrow, 0)
        for r in range(RPT // ZR):
            pltpu.sync_copy(zbuf, acc.at[pl.ds(s * RPT + r * ZR, ZR)])
        plsc.subcore_barrier()

        pltpu.sync_copy(srcg_hbm.at[wid], srcg_v)
        pltpu.sync_copy(dstg_hbm.at[wid], dstg_v)
        pltpu.sync_copy(dstl_hbm.at[s], dstl_v)

        def chunk(j, _):
            ga = pltpu.async_copy(a_hbm.at[dstg_v.at[j]], a_v, sem_a)
            gb = pltpu.async_copy(b_hbm.at[srcg_v.at[j]], b_v, sem_b)
            ga.wait()
            gb.wait()

            def erow(e, _):
                for q in range(H // L):
                    v = a_v[e, pl.ds(q * L, L)] + b_v[e, pl.ds(q * L, L)]
                    g_v[e, pl.ds(q * L, L)] = (
                        jnp.maximum(v, 0.0) + 0.2 * jnp.minimum(v, 0.0))
                return 0
            lax.fori_loop(0, CHUNK, erow, 0)
            pltpu.sync_copy(g_v, acc.at[dstl_v.at[j]], add=True)
            return 0
        lax.fori_loop(0, CPT, chunk, 0)
        plsc.subcore_barrier()

        pltpu.sync_copy(acc.at[pl.ds(s * RPT, RPT)],
                        out_hbm.at[pl.ds(c * NP + s * RPT, RPT)])

    return mp_kernel


@functools.lru_cache(maxsize=None)
def _make_deg_sc(NP, CPT):
    RPT = NP // NS
    HCPT = _cdiv(CPT, NC)
    mesh = plsc.VectorSubcoreMesh(core_axis_name="c", subcore_axis_name="s")

    @functools.partial(
        pl.kernel,
        out_type=jax.ShapeDtypeStruct((NC * NP, L), jnp.float32),
        mesh=mesh,
        compiler_params=pltpu.CompilerParams(use_tc_tiling_on_sc=False),
        scratch_types=[
            pltpu.VMEM((CPT, CHUNK), jnp.int32),
            pltpu.VMEM((CHUNK, L), jnp.float32),
            pltpu.VMEM_SHARED((NP, L), jnp.float32),
        ],
    )
    def deg_kernel(dstl_hbm, out_hbm, dstl_v, ones_v, accd):
        c = lax.axis_index("c")
        s = lax.axis_index("s")
        zero = jnp.zeros((L,), jnp.float32)
        one = jnp.full((L,), 1.0, jnp.float32)

        def zdrow(i, _):
            ones_v[i, pl.ds(0, L)] = zero
            return 0
        lax.fori_loop(0, CHUNK, zdrow, 0)
        for r in range(RPT // CHUNK):
            pltpu.sync_copy(ones_v, accd.at[pl.ds(s * RPT + r * CHUNK, CHUNK)])

        def ones_row(e, _):
            ones_v[e, pl.ds(0, L)] = one
            return 0
        lax.fori_loop(0, CHUNK, ones_row, 0)
        plsc.subcore_barrier()

        pltpu.sync_copy(dstl_hbm.at[s], dstl_v)

        def chunk(jj, _):
            j = c * HCPT + jj

            @pl.when(j < CPT)
            def _():
                pltpu.sync_copy(ones_v, accd.at[dstl_v.at[j]], add=True)
            return 0
        lax.fori_loop(0, HCPT, chunk, 0)
        plsc.subcore_barrier()

        pltpu.sync_copy(accd.at[pl.ds(s * RPT, RPT)],
                        out_hbm.at[pl.ds(c * NP + s * RPT, RPT)])

    return deg_kernel


@functools.lru_cache(maxsize=None)
def _make_hop_sc(NP, H, CPT):
    RPT = NP // NS
    mesh = plsc.VectorSubcoreMesh(core_axis_name="c", subcore_axis_name="s")

    @functools.partial(
        pl.kernel,
        out_type=jax.ShapeDtypeStruct((NC * NP, H), jnp.float32),
        mesh=mesh,
        compiler_params=pltpu.CompilerParams(use_tc_tiling_on_sc=False),
        scratch_types=[
            pltpu.VMEM((CPT, CHUNK), jnp.int32),
            pltpu.VMEM((CPT, CHUNK), jnp.int32),
            pltpu.VMEM((CHUNK, H), jnp.float32),
            pltpu.VMEM((ZR, H), jnp.float32),
            pltpu.VMEM_SHARED((NP, H), jnp.float32),
            pltpu.SemaphoreType.DMA,
        ],
    )
    def hop_kernel(z_hbm, srcg_hbm, dstl_hbm, out_hbm,
                   srcg_v, dstl_v, rows_v, zbuf, acc, sem):
        c = lax.axis_index("c")
        s = lax.axis_index("s")
        wid = c * NS + s
        zero = jnp.zeros((L,), jnp.float32)

        def zrow(i, _):
            for q in range(H // L):
                zbuf[i, pl.ds(q * L, L)] = zero
            return 0
        lax.fori_loop(0, ZR, zrow, 0)
        for r in range(RPT // ZR):
            pltpu.sync_copy(zbuf, acc.at[pl.ds(s * RPT + r * ZR, ZR)])
        plsc.subcore_barrier()

        pltpu.sync_copy(srcg_hbm.at[wid], srcg_v)
        pltpu.sync_copy(dstl_hbm.at[s], dstl_v)

        def chunk(j, _):
            pltpu.async_copy(z_hbm.at[srcg_v.at[j]], rows_v, sem).wait()
            pltpu.sync_copy(rows_v, acc.at[dstl_v.at[j]], add=True)
            return 0
        lax.fori_loop(0, CPT, chunk, 0)
        plsc.subcore_barrier()

        pltpu.sync_copy(acc.at[pl.ds(s * RPT, RPT)],
                        out_hbm.at[pl.ds(c * NP + s * RPT, RPT)])

    return hop_kernel


BR = 1024


def _dot_t(a, w):
    return lax.dot_general(a, w, (((1,), (1,)), ((), ())),
                           preferred_element_type=jnp.float32)


def _prep_body(x_ref, w1_ref, b1_ref, a_ref, b_ref):
    x = x_ref[...]
    w = w1_ref[...]
    d = x.shape[1]
    a_ref[...] = _dot_t(x, w[:, :d]) + b1_ref[...]
    b_ref[...] = _dot_t(x, w[:, d:])


def _postmp_body(p0_ref, p1_ref, d0_ref, d1_ref, w2_ref, b2_ref, w00_ref,
                 acc_ref, z_ref, dinv_ref):
    ssum = jnp.concatenate([p0_ref[...], p1_ref[...]], axis=1)
    deg = d0_ref[...][:, :1] + d1_ref[...][:, :1]
    x1 = _dot_t(ssum, w2_ref[...]) + deg * b2_ref[...]
    dinv = jnp.where(deg > 0, lax.rsqrt(jnp.maximum(deg, 1e-12)), 0.0)
    acc_ref[...] = _dot_t(x1, w00_ref[...])
    z_ref[...] = dinv * x1
    dinv_ref[...] = dinv


def _hop_mid_body(p0_ref, p1_ref, dinv_ref, acc_ref, wk_ref,
                  acc_out, z_out):
    dinv = dinv_ref[...]
    h = dinv * jnp.concatenate([p0_ref[...], p1_ref[...]], axis=1)
    acc_out[...] = acc_ref[...] + _dot_t(h, wk_ref[...])
    z_out[...] = dinv * h


def _hop_end_body(p0_ref, p1_ref, dinv_ref, acc_ref, wk_ref, bl_ref, wn_ref,
                  acc_out, z_out):
    dinv = dinv_ref[...]
    h = dinv * jnp.concatenate([p0_ref[...], p1_ref[...]], axis=1)
    xn = jnp.maximum(acc_ref[...] + _dot_t(h, wk_ref[...]) + bl_ref[...], 0.0)
    acc_out[...] = _dot_t(xn, wn_ref[...])
    z_out[...] = dinv * xn


def _hop_fin_body(p0_ref, p1_ref, dinv_ref, acc_ref, wk_ref, bl_ref,
                  xn_out):
    dinv = dinv_ref[...]
    h = dinv * jnp.concatenate([p0_ref[...], p1_ref[...]], axis=1)
    xn_out[...] = jnp.maximum(
        acc_ref[...] + _dot_t(h, wk_ref[...]) + bl_ref[...], 0.0)


def _row_spec(shape):
    return pl.BlockSpec((BR,) + shape[1:], lambda i: (i,) + (0,) * (len(shape) - 1))


def _full_spec(shape):
    n = len(shape)
    return pl.BlockSpec(shape, lambda i: (0,) * n)


def _shift_spec(shape, off):
    return pl.BlockSpec((BR,) + shape[1:], lambda i, o=off: (i + o,) + (0,) * (len(shape) - 1))


def _tc_call(body, grid, in_arrays, in_specs, out_shapes):
    out_specs = [_row_spec(s.shape) for s in out_shapes]
    return pl.pallas_call(
        body,
        grid=(grid,),
        in_specs=in_specs,
        out_specs=out_specs if len(out_specs) > 1 else out_specs[0],
        out_shape=out_shapes if len(out_shapes) > 1 else out_shapes[0],
    )(*in_arrays)


def kernel(x, edge_index, mp_w1, mp_b1, mp_w2, mp_b2, tag_ws, tag_bs):
    N, D = x.shape
    E = edge_index.shape[1]
    K_CONVS, KH1 = tag_ws.shape[0], tag_ws.shape[1]
    K_HOPS = KH1 - 1
    H = D // NC

    NP = _cdiv(N + 1, NS * ZR) * NS * ZR
    CPT = _cdiv(E, NS * CHUNK)
    EP = NS * CPT * CHUNK
    nblk = NP // BR

    xp = jnp.zeros((NP, D), jnp.float32).at[:N].set(x)
    pad = EP - E
    srcp = jnp.concatenate(
        [edge_index[0], jnp.full((pad,), N, jnp.int32)]).reshape(NS, CPT, CHUNK)
    dstp = jnp.concatenate(
        [edge_index[1], jnp.full((pad,), N, jnp.int32)]).reshape(NS, CPT, CHUNK)
    srcg = jnp.concatenate([2 * srcp[None], 2 * srcp[None] + 1], axis=0)
    dstg = jnp.concatenate([2 * dstp[None], 2 * dstp[None] + 1], axis=0)
    srcg = srcg.reshape(NW, CPT, CHUNK)
    dstg = dstg.reshape(NW, CPT, CHUNK)

    f32 = jnp.float32
    sds = jax.ShapeDtypeStruct

    A, B = _tc_call(
        _prep_body, nblk,
        [xp, mp_w1, mp_b1[None]],
        [_row_spec((NP, D)), _full_spec(mp_w1.shape), _full_spec((1, D))],
        [sds((NP, D), f32), sds((NP, D), f32)])

    P = _make_mp_sc(NP, H, CPT)(
        A.reshape(NC * NP, H), B.reshape(NC * NP, H), srcg, dstg, dstp)
    deg = _make_deg_sc(NP, CPT)(dstp)

    acc, z, dinv = _tc_call(
        _postmp_body, nblk,
        [P, P, deg, deg, mp_w2, mp_b2[None], tag_ws[0, 0]],
        [_row_spec((NC * NP, H)), _shift_spec((NC * NP, H), nblk),
         _row_spec((NC * NP, L)), _shift_spec((NC * NP, L), nblk),
         _full_spec(mp_w2.shape), _full_spec((1, D)), _full_spec((D, D))],
        [sds((NP, D), f32), sds((NP, D), f32), sds((NP, 1), f32)])

    hop_sc = _make_hop_sc(NP, H, CPT)
    xn = None
    for l in range(K_CONVS):
        for k in range(1, K_HOPS + 1):
            Ph = hop_sc(z.reshape(NC * NP, H), srcg, dstp)
            p_specs = [_row_spec((NC * NP, H)), _shift_spec((NC * NP, H), nblk),
                       _row_spec((NP, 1)), _row_spec((NP, D)),
                       _full_spec((D, D))]
            if k < K_HOPS:
                acc, z = _tc_call(
                    _hop_mid_body, nblk,
                    [Ph, Ph, dinv, acc, tag_ws[l, k]],
                    p_specs,
                    [sds((NP, D), f32), sds((NP, D), f32)])
            elif l < K_CONVS - 1:
                acc, z = _tc_call(
                    _hop_end_body, nblk,
                    [Ph, Ph, dinv, acc, tag_ws[l, k], tag_bs[l][None],
                     tag_ws[l + 1, 0]],
                    p_specs + [_full_spec((1, D)), _full_spec((D, D))],
                    [sds((NP, D), f32), sds((NP, D), f32)])
            else:
                xn = _tc_call(
                    _hop_fin_body, nblk,
                    [Ph, Ph, dinv, acc, tag_ws[l, k], tag_bs[l][None]],
                    p_specs + [_full_spec((1, D))],
                    [sds((NP, D), f32)])

    return xn[:N]


# speedup vs baseline: 6.7009x; 1.0192x over previous
import functools

import jax
import jax.numpy as jnp
from jax import lax
from jax.experimental import pallas as pl
from jax.experimental.pallas import tpu as pltpu
from jax.experimental.pallas import tpu_sc as plsc

NC = 2
NS = 16
NW = NC * NS
L = 16
CHUNK = 128
ZR = 64


def _cdiv(a, b):
    return -(-a // b)


@functools.lru_cache(maxsize=None)
def _make_mp_sc(NP, H, CPT):
    RPT = NP // NS
    mesh = plsc.VectorSubcoreMesh(core_axis_name="c", subcore_axis_name="s")

    @functools.partial(
        pl.kernel,
        out_type=jax.ShapeDtypeStruct((NC * NP, H), jnp.float32),
        mesh=mesh,
        compiler_params=pltpu.CompilerParams(use_tc_tiling_on_sc=False),
        scratch_types=[
            pltpu.VMEM((CPT, CHUNK), jnp.int32),
            pltpu.VMEM((CPT, CHUNK), jnp.int32),
            pltpu.VMEM((CPT, CHUNK), jnp.int32),
            pltpu.VMEM((CHUNK, H), jnp.float32),
            pltpu.VMEM((CHUNK, H), jnp.float32),
            pltpu.VMEM((CHUNK, H), jnp.float32),
            pltpu.VMEM((ZR, H), jnp.float32),
            pltpu.VMEM_SHARED((NP, H), jnp.float32),
            pltpu.SemaphoreType.DMA,
            pltpu.SemaphoreType.DMA,
        ],
    )
    def mp_kernel(a_hbm, b_hbm, srcg_hbm, dstg_hbm, dstl_hbm, out_hbm,
                  srcg_v, dstg_v, dstl_v, a_v, b_v, g_v, zbuf,
                  acc, sem_a, sem_b):
        c = lax.axis_index("c")
        s = lax.axis_index("s")
        wid = c * NS + s
        zero = jnp.zeros((L,), jnp.float32)

        def zrow(i, _):
            for q in range(H // L):
                zbuf[i, pl.ds(q * L, L)] = zero
            return 0
        lax.fori_loop(0, ZR, zrow, 0)
        for r in range(RPT // ZR):
            pltpu.sync_copy(zbuf, acc.at[pl.ds(s * RPT + r * ZR, ZR)])
        plsc.subcore_barrier()

        pltpu.sync_copy(srcg_hbm.at[wid], srcg_v)
        pltpu.sync_copy(dstg_hbm.at[wid], dstg_v)
        pltpu.sync_copy(dstl_hbm.at[s], dstl_v)

        def chunk(j, _):
            ga = pltpu.async_copy(a_hbm.at[dstg_v.at[j]], a_v, sem_a)
            gb = pltpu.async_copy(b_hbm.at[srcg_v.at[j]], b_v, sem_b)
            ga.wait()
            gb.wait()

            def erow(e, _):
                for q in range(H // L):
                    v = a_v[e, pl.ds(q * L, L)] + b_v[e, pl.ds(q * L, L)]
                    g_v[e, pl.ds(q * L, L)] = (
                        jnp.maximum(v, 0.0) + 0.2 * jnp.minimum(v, 0.0))
                return 0
            lax.fori_loop(0, CHUNK, erow, 0)
            pltpu.sync_copy(g_v, acc.at[dstl_v.at[j]], add=True)
            return 0
        lax.fori_loop(0, CPT, chunk, 0)
        plsc.subcore_barrier()

        pltpu.sync_copy(acc.at[pl.ds(s * RPT, RPT)],
                        out_hbm.at[pl.ds(c * NP + s * RPT, RPT)])

    return mp_kernel


@functools.lru_cache(maxsize=None)
def _make_deg_sc(NP, CPT):
    RPT = NP // NS
    HCPT = _cdiv(CPT, NC)
    mesh = plsc.VectorSubcoreMesh(core_axis_name="c", subcore_axis_name="s")

    @functools.partial(
        pl.kernel,
        out_type=jax.ShapeDtypeStruct((NC * NP, L), jnp.float32),
        mesh=mesh,
        compiler_params=pltpu.CompilerParams(use_tc_tiling_on_sc=False),
        scratch_types=[
            pltpu.VMEM((CPT, CHUNK), jnp.int32),
            pltpu.VMEM((CHUNK, L), jnp.float32),
            pltpu.VMEM_SHARED((NP, L), jnp.float32),
        ],
    )
    def deg_kernel(dstl_hbm, out_hbm, dstl_v, ones_v, accd):
        c = lax.axis_index("c")
        s = lax.axis_index("s")
        zero = jnp.zeros((L,), jnp.float32)
        one = jnp.full((L,), 1.0, jnp.float32)

        def zdrow(i, _):
            ones_v[i, pl.ds(0, L)] = zero
            return 0
        lax.fori_loop(0, CHUNK, zdrow, 0)
        for r in range(RPT // CHUNK):
            pltpu.sync_copy(ones_v, accd.at[pl.ds(s * RPT + r * CHUNK, CHUNK)])

        def ones_row(e, _):
            ones_v[e, pl.ds(0, L)] = one
            return 0
        lax.fori_loop(0, CHUNK, ones_row, 0)
        plsc.subcore_barrier()

        pltpu.sync_copy(dstl_hbm.at[s], dstl_v)

        def chunk(jj, _):
            j = c * HCPT + jj

            @pl.when(j < CPT)
            def _():
                pltpu.sync_copy(ones_v, accd.at[dstl_v.at[j]], add=True)
            return 0
        lax.fori_loop(0, HCPT, chunk, 0)
        plsc.subcore_barrier()

        pltpu.sync_copy(accd.at[pl.ds(s * RPT, RPT)],
                        out_hbm.at[pl.ds(c * NP + s * RPT, RPT)])

    return deg_kernel


@functools.lru_cache(maxsize=None)
def _make_hop_sc(NP, H, CPT):
    RPT = NP // NS
    mesh = plsc.VectorSubcoreMesh(core_axis_name="c", subcore_axis_name="s")

    @functools.partial(
        pl.kernel,
        out_type=jax.ShapeDtypeStruct((NC * NP, H), jnp.float32),
        mesh=mesh,
        compiler_params=pltpu.CompilerParams(use_tc_tiling_on_sc=False),
        scratch_types=[
            pltpu.VMEM((CPT, CHUNK), jnp.int32),
            pltpu.VMEM((CPT, CHUNK), jnp.int32),
            pltpu.VMEM((2, CHUNK, H), jnp.float32),
            pltpu.VMEM((ZR, H), jnp.float32),
            pltpu.VMEM_SHARED((NP, H), jnp.float32),
            pltpu.SemaphoreType.DMA,
            pltpu.SemaphoreType.DMA,
        ],
    )
    def hop_kernel(z_hbm, srcg_hbm, dstl_hbm, out_hbm,
                   srcg_v, dstl_v, rows_v, zbuf, acc, sem0, sem1):
        c = lax.axis_index("c")
        s = lax.axis_index("s")
        wid = c * NS + s
        zero = jnp.zeros((L,), jnp.float32)

        def zrow(i, _):
            for q in range(H // L):
                zbuf[i, pl.ds(q * L, L)] = zero
            return 0
        lax.fori_loop(0, ZR, zrow, 0)
        for r in range(RPT // ZR):
            pltpu.sync_copy(zbuf, acc.at[pl.ds(s * RPT + r * ZR, ZR)])
        plsc.subcore_barrier()

        pltpu.sync_copy(srcg_hbm.at[wid], srcg_v)
        pltpu.sync_copy(dstl_hbm.at[s], dstl_v)

        sems = (sem0, sem1)

        pltpu.async_copy(z_hbm.at[srcg_v.at[0]], rows_v.at[0], sem0)

        def group(g, _):
            for b in range(2):
                j = 2 * g + b
                nb = 1 - b

                @pl.when(j + 1 < CPT)
                def _():
                    pltpu.async_copy(
                        z_hbm.at[srcg_v.at[j + 1]], rows_v.at[nb], sems[nb])
                pltpu.make_async_copy(
                    z_hbm.at[srcg_v.at[j]], rows_v.at[b], sems[b]).wait()
                pltpu.sync_copy(rows_v.at[b], acc.at[dstl_v.at[j]], add=True)
            return 0
        lax.fori_loop(0, CPT // 2, group, 0)
        plsc.subcore_barrier()

        pltpu.sync_copy(acc.at[pl.ds(s * RPT, RPT)],
                        out_hbm.at[pl.ds(c * NP + s * RPT, RPT)])

    return hop_kernel


BR = 1024


def _dot_t(a, w):
    return lax.dot_general(a, w, (((1,), (1,)), ((), ())),
                           preferred_element_type=jnp.float32)


def _prep_body(x_ref, w1_ref, b1_ref, a_ref, b_ref):
    x = x_ref[...]
    w = w1_ref[...]
    d = x.shape[1]
    a_ref[...] = _dot_t(x, w[:, :d]) + b1_ref[...]
    b_ref[...] = _dot_t(x, w[:, d:])


def _postmp_body(p0_ref, p1_ref, d0_ref, d1_ref, w2_ref, b2_ref, w00_ref,
                 acc_ref, z_ref, dinv_ref):
    ssum = jnp.concatenate([p0_ref[...], p1_ref[...]], axis=1)
    deg = d0_ref[...][:, :1] + d1_ref[...][:, :1]
    x1 = _dot_t(ssum, w2_ref[...]) + deg * b2_ref[...]
    dinv = jnp.where(deg > 0, lax.rsqrt(jnp.maximum(deg, 1e-12)), 0.0)
    acc_ref[...] = _dot_t(x1, w00_ref[...])
    z_ref[...] = dinv * x1
    dinv_ref[...] = dinv


def _hop_mid_body(p0_ref, p1_ref, dinv_ref, acc_ref, wk_ref,
                  acc_out, z_out):
    dinv = dinv_ref[...]
    h = dinv * jnp.concatenate([p0_ref[...], p1_ref[...]], axis=1)
    acc_out[...] = acc_ref[...] + _dot_t(h, wk_ref[...])
    z_out[...] = dinv * h


def _hop_end_body(p0_ref, p1_ref, dinv_ref, acc_ref, wk_ref, bl_ref, wn_ref,
                  acc_out, z_out):
    dinv = dinv_ref[...]
    h = dinv * jnp.concatenate([p0_ref[...], p1_ref[...]], axis=1)
    xn = jnp.maximum(acc_ref[...] + _dot_t(h, wk_ref[...]) + bl_ref[...], 0.0)
    acc_out[...] = _dot_t(xn, wn_ref[...])
    z_out[...] = dinv * xn


def _hop_fin_body(p0_ref, p1_ref, dinv_ref, acc_ref, wk_ref, bl_ref,
                  xn_out):
    dinv = dinv_ref[...]
    h = dinv * jnp.concatenate([p0_ref[...], p1_ref[...]], axis=1)
    xn_out[...] = jnp.maximum(
        acc_ref[...] + _dot_t(h, wk_ref[...]) + bl_ref[...], 0.0)


def _row_spec(shape):
    return pl.BlockSpec((BR,) + shape[1:], lambda i: (i,) + (0,) * (len(shape) - 1))


def _full_spec(shape):
    n = len(shape)
    return pl.BlockSpec(shape, lambda i: (0,) * n)


def _shift_spec(shape, off):
    return pl.BlockSpec((BR,) + shape[1:], lambda i, o=off: (i + o,) + (0,) * (len(shape) - 1))


def _tc_call(body, grid, in_arrays, in_specs, out_shapes):
    out_specs = [_row_spec(s.shape) for s in out_shapes]
    return pl.pallas_call(
        body,
        grid=(grid,),
        in_specs=in_specs,
        out_specs=out_specs if len(out_specs) > 1 else out_specs[0],
        out_shape=out_shapes if len(out_shapes) > 1 else out_shapes[0],
    )(*in_arrays)


def kernel(x, edge_index, mp_w1, mp_b1, mp_w2, mp_b2, tag_ws, tag_bs):
    N, D = x.shape
    E = edge_index.shape[1]
    K_CONVS, KH1 = tag_ws.shape[0], tag_ws.shape[1]
    K_HOPS = KH1 - 1
    H = D // NC

    NP = _cdiv(N + 1, NS * ZR) * NS * ZR
    CPT = 2 * _cdiv(E, NS * CHUNK * 2)
    EP = NS * CPT * CHUNK
    nblk = NP // BR

    xp = jnp.zeros((NP, D), jnp.float32).at[:N].set(x)
    pad = EP - E
    srcp = jnp.concatenate(
        [edge_index[0], jnp.full((pad,), N, jnp.int32)]).reshape(NS, CPT, CHUNK)
    dstp = jnp.concatenate(
        [edge_index[1], jnp.full((pad,), N, jnp.int32)]).reshape(NS, CPT, CHUNK)
    srcg = jnp.concatenate([2 * srcp[None], 2 * srcp[None] + 1], axis=0)
    dstg = jnp.concatenate([2 * dstp[None], 2 * dstp[None] + 1], axis=0)
    srcg = srcg.reshape(NW, CPT, CHUNK)
    dstg = dstg.reshape(NW, CPT, CHUNK)

    f32 = jnp.float32
    sds = jax.ShapeDtypeStruct

    A, B = _tc_call(
        _prep_body, nblk,
        [xp, mp_w1, mp_b1[None]],
        [_row_spec((NP, D)), _full_spec(mp_w1.shape), _full_spec((1, D))],
        [sds((NP, D), f32), sds((NP, D), f32)])

    P = _make_mp_sc(NP, H, CPT)(
        A.reshape(NC * NP, H), B.reshape(NC * NP, H), srcg, dstg, dstp)
    deg = _make_deg_sc(NP, CPT)(dstp)

    acc, z, dinv = _tc_call(
        _postmp_body, nblk,
        [P, P, deg, deg, mp_w2, mp_b2[None], tag_ws[0, 0]],
        [_row_spec((NC * NP, H)), _shift_spec((NC * NP, H), nblk),
         _row_spec((NC * NP, L)), _shift_spec((NC * NP, L), nblk),
         _full_spec(mp_w2.shape), _full_spec((1, D)), _full_spec((D, D))],
        [sds((NP, D), f32), sds((NP, D), f32), sds((NP, 1), f32)])

    hop_sc = _make_hop_sc(NP, H, CPT)
    xn = None
    for l in range(K_CONVS):
        for k in range(1, K_HOPS + 1):
            Ph = hop_sc(z.reshape(NC * NP, H), srcg, dstp)
            p_specs = [_row_spec((NC * NP, H)), _shift_spec((NC * NP, H), nblk),
                       _row_spec((NP, 1)), _row_spec((NP, D)),
                       _full_spec((D, D))]
            if k < K_HOPS:
                acc, z = _tc_call(
                    _hop_mid_body, nblk,
                    [Ph, Ph, dinv, acc, tag_ws[l, k]],
                    p_specs,
                    [sds((NP, D), f32), sds((NP, D), f32)])
            elif l < K_CONVS - 1:
                acc, z = _tc_call(
                    _hop_end_body, nblk,
                    [Ph, Ph, dinv, acc, tag_ws[l, k], tag_bs[l][None],
                     tag_ws[l + 1, 0]],
                    p_specs + [_full_spec((1, D)), _full_spec((D, D))],
                    [sds((NP, D), f32), sds((NP, D), f32)])
            else:
                xn = _tc_call(
                    _hop_fin_body, nblk,
                    [Ph, Ph, dinv, acc, tag_ws[l, k], tag_bs[l][None]],
                    p_specs + [_full_spec((1, D))],
                    [sds((NP, D), f32)])

    return xn[:N]
